# baseline (speedup 1.0000x reference)
import functools

import jax
import jax.numpy as jnp
from jax import lax
from jax.experimental import pallas as pl
from jax.experimental.pallas import tpu as pltpu
from jax.experimental.pallas import tpu_sc as plsc

_N_CELLS = 50000
_N_GENES = 128
_N_LABELS = 8
_EMB = 64
_B = 16384

_BT = 2048


def _make_sc_gather():
    info = plsc.get_sparse_core_info()
    nc, ns, nl = info.num_cores, info.num_subcores, info.num_lanes
    nw = nc * ns
    bw = _B // nw

    mesh = plsc.VectorSubcoreMesh(core_axis_name="c", subcore_axis_name="s")

    @functools.partial(
        pl.kernel,
        mesh=mesh,
        compiler_params=pltpu.CompilerParams(use_tc_tiling_on_sc=False),
        out_type=[
            jax.ShapeDtypeStruct((_B, _EMB), jnp.float32),
            jax.ShapeDtypeStruct((_B, _N_LABELS), jnp.float32),
        ],
        scratch_types=[
            pltpu.VMEM((bw,), jnp.int32),
            pltpu.VMEM((bw,), jnp.int32),
            pltpu.VMEM((bw,), jnp.int32),
            pltpu.VMEM((bw, _EMB), jnp.float32),
            pltpu.VMEM((bw, _N_LABELS), jnp.float32),
            pltpu.SemaphoreType.DMA,
        ],
    )
    def sc_kernel(cells_hbm, genes_hbm, emb_hbm, pzcw_hbm,
                  emb_out, qz_out,
                  cells_v, genes_v, idx_v, emb_v, qz_v, sem):
        wid = lax.axis_index("s") * nc + lax.axis_index("c")
        base = wid * bw

        pltpu.sync_copy(cells_hbm.at[pl.ds(base, bw)], cells_v)
        pltpu.sync_copy(genes_hbm.at[pl.ds(base, bw)], genes_v)

        cp_emb = pltpu.async_copy(emb_hbm.at[cells_v], emb_v, sem)

        def idx_body(k, _):
            c = cells_v[pl.ds(k * nl, nl)]
            g = genes_v[pl.ds(k * nl, nl)]
            idx_v[pl.ds(k * nl, nl)] = c * _N_GENES + g
            return 0
        lax.fori_loop(0, bw // nl, idx_body, 0, unroll=4)

        cp_qz = pltpu.async_copy(pzcw_hbm.at[idx_v], qz_v, sem)

        cp_emb.wait()
        pltpu.sync_copy(emb_v, emb_out.at[pl.ds(base, bw)])
        cp_qz.wait()
        pltpu.sync_copy(qz_v, qz_out.at[pl.ds(base, bw)])

    return sc_kernel


def _tc_body(genes_ref, emb_ref, ctw_ref, decw_ref, decb_ref,
             pcz_ref, pz_ref):
    j = pl.program_id(0)
    ctw = ctw_ref[...]

    logits = lax.dot_general(
        emb_ref[...], ctw, (((1,), (1,)), ((), ())),
        preferred_element_type=jnp.float32)
    m = jnp.max(logits, axis=1, keepdims=True)
    e = jnp.exp(logits - m)
    pz = e / jnp.sum(e, axis=1, keepdims=True)
    pz_ref[...] = pz

    dec = lax.dot_general(
        decw_ref[...], ctw, (((1,), (1,)), ((), ())),
        preferred_element_type=jnp.float32)
    dec = dec + decb_ref[...].reshape(_N_GENES, 1)
    m0 = jnp.max(dec, axis=0, keepdims=True)
    e0 = jnp.exp(dec - m0)
    pc_zw = e0 / jnp.sum(e0, axis=0, keepdims=True)

    g = genes_ref[0, pl.ds(j * _BT, _BT)]
    oh = (g[:, None] == lax.broadcasted_iota(jnp.int32, (_BT, _N_GENES), 1))
    pc_rows = lax.dot_general(
        oh.astype(jnp.float32), pc_zw, (((1,), (0,)), ((), ())),
        preferred_element_type=jnp.float32)
    pcz_ref[...] = pz * pc_rows


def _tc_finish(genes, emb_rows, ct_W, dec_W, dec_b):
    return pl.pallas_call(
        _tc_body,
        grid=(_B // _BT,),
        in_specs=[
            pl.BlockSpec((1, _B), lambda j: (0, 0)),
            pl.BlockSpec((_BT, _EMB), lambda j: (j, 0)),
            pl.BlockSpec((_N_LABELS, _EMB), lambda j: (0, 0)),
            pl.BlockSpec((_N_GENES, _EMB), lambda j: (0, 0)),
            pl.BlockSpec((1, _N_GENES), lambda j: (0, 0)),
        ],
        out_specs=[
            pl.BlockSpec((_BT, _N_LABELS), lambda j: (j, 0)),
            pl.BlockSpec((_BT, _N_LABELS), lambda j: (j, 0)),
        ],
        out_shape=[
            jax.ShapeDtypeStruct((_B, _N_LABELS), jnp.float32),
            jax.ShapeDtypeStruct((_B, _N_LABELS), jnp.float32),
        ],
    )(genes.reshape(1, _B), emb_rows, ct_W, dec_W, dec_b.reshape(1, _N_GENES))


def kernel(cells, genes, emb_table, ct_W, dec_W, dec_b, pz_cw):
    cells = cells.astype(jnp.int32)
    genes = genes.astype(jnp.int32)
    pzcw_flat = pz_cw.reshape(_N_CELLS * _N_GENES, _N_LABELS)
    sc = _make_sc_gather()
    emb_rows, qz_w = sc(cells, genes, emb_table, pzcw_flat)
    pcz_w, pz_w = _tc_finish(genes, emb_rows, ct_W, dec_W, dec_b)
    return (pcz_w, qz_w, pz_w)


# speedup vs baseline: 15.6818x; 15.6818x over previous
import functools

import jax
import jax.numpy as jnp
from jax import lax
from jax.experimental import pallas as pl
from jax.experimental.pallas import tpu as pltpu
from jax.experimental.pallas import tpu_sc as plsc

_N_CELLS = 50000
_N_GENES = 128
_N_LABELS = 8
_EMB = 64
_B = 16384

_BKC = 4096


def _tc_tables_body(embt_ref, ctw_ref, decw_ref, decb_ref, p_ref, pct_ref):
    ctw = ctw_ref[...]
    logits = lax.dot_general(
        embt_ref[...], ctw, (((0,), (1,)), ((), ())),
        preferred_element_type=jnp.float32)
    m = jnp.max(logits, axis=1, keepdims=True)
    e = jnp.exp(logits - m)
    p_ref[...] = e / jnp.sum(e, axis=1, keepdims=True)

    @pl.when(pl.program_id(0) == 0)
    def _():
        dec = lax.dot_general(
            ctw, decw_ref[...], (((1,), (1,)), ((), ())),
            preferred_element_type=jnp.float32)
        dec = dec + decb_ref[...]
        m0 = jnp.max(dec, axis=1, keepdims=True)
        e0 = jnp.exp(dec - m0)
        pct_ref[...] = e0 / jnp.sum(e0, axis=1, keepdims=True)


def _tc_tables(emb_t, ct_W, dec_W, dec_b):
    grid = (_N_CELLS + _BKC - 1) // _BKC
    return pl.pallas_call(
        _tc_tables_body,
        grid=(grid,),
        in_specs=[
            pl.BlockSpec((_EMB, _BKC), lambda i: (0, i)),
            pl.BlockSpec((_N_LABELS, _EMB), lambda i: (0, 0)),
            pl.BlockSpec((_N_GENES, _EMB), lambda i: (0, 0)),
            pl.BlockSpec((1, _N_GENES), lambda i: (0, 0)),
        ],
        out_specs=[
            pl.BlockSpec((_BKC, _N_LABELS), lambda i: (i, 0)),
            pl.BlockSpec((_N_LABELS, _N_GENES), lambda i: (0, 0)),
        ],
        out_shape=[
            jax.ShapeDtypeStruct((_N_CELLS, _N_LABELS), jnp.float32),
            jax.ShapeDtypeStruct((_N_LABELS, _N_GENES), jnp.float32),
        ],
    )(emb_t, ct_W, dec_W, dec_b.reshape(1, _N_GENES))


def _make_sc_gather():
    info = plsc.get_sparse_core_info()
    nc, ns, nl = info.num_cores, info.num_subcores, info.num_lanes
    nw = nc * ns
    bw = _B // nw
    nlb = _N_LABELS

    mesh = plsc.VectorSubcoreMesh(core_axis_name="c", subcore_axis_name="s")

    scratch = (
        [pltpu.VMEM((bw,), jnp.int32)] * 2
        + [pltpu.VMEM((bw,), jnp.int32)] * (3 * nlb)
        + [pltpu.VMEM((bw,), jnp.float32)] * (4 * nlb)
        + [pltpu.SemaphoreType.DMA]
    )

    @functools.partial(
        pl.kernel,
        mesh=mesh,
        compiler_params=pltpu.CompilerParams(use_tc_tiling_on_sc=False),
        out_type=[
            jax.ShapeDtypeStruct((nlb, _B), jnp.float32),
            jax.ShapeDtypeStruct((nlb, _B), jnp.float32),
            jax.ShapeDtypeStruct((nlb, _B), jnp.float32),
        ],
        scratch_types=scratch,
    )
    def sc_kernel(cells_hbm, genes_hbm, pzcw_hbm, pflat_hbm, pcflat_hbm,
                  pcz_out, qz_out, pz_out, *refs):
        cells_v, genes_v = refs[0], refs[1]
        qidx = refs[2:2 + nlb]
        pidx = refs[2 + nlb:2 + 2 * nlb]
        cidx = refs[2 + 2 * nlb:2 + 3 * nlb]
        qzv = refs[2 + 3 * nlb:2 + 4 * nlb]
        pzv = refs[2 + 4 * nlb:2 + 5 * nlb]
        pcv = refs[2 + 5 * nlb:2 + 6 * nlb]
        pczv = refs[2 + 6 * nlb:2 + 7 * nlb]
        sem = refs[2 + 7 * nlb]

        wid = lax.axis_index("s") * nc + lax.axis_index("c")
        base = wid * bw

        pltpu.sync_copy(cells_hbm.at[pl.ds(base, bw)], cells_v)
        pltpu.sync_copy(genes_hbm.at[pl.ds(base, bw)], genes_v)

        def idx_body(k, _):
            sl = pl.ds(k * nl, nl)
            c = cells_v[sl]
            g = genes_v[sl]
            qb = c * 1024 + g
            pb = c * nlb
            cb = g * nlb
            for l in range(nlb):
                qidx[l][sl] = qb + l * 128
                pidx[l][sl] = pb + l
                cidx[l][sl] = cb + l
            return 0
        lax.fori_loop(0, bw // nl, idx_body, 0)

        copies = []
        for l in range(nlb):
            copies.append(pltpu.async_copy(pzcw_hbm.at[qidx[l]], qzv[l], sem))
            copies.append(pltpu.async_copy(pflat_hbm.at[pidx[l]], pzv[l], sem))
            copies.append(pltpu.async_copy(pcflat_hbm.at[cidx[l]], pcv[l], sem))
        for cp in copies:
            cp.wait()

        def mul_body(k, _):
            sl = pl.ds(k * nl, nl)
            for l in range(nlb):
                pczv[l][sl] = pzv[l][sl] * pcv[l][sl]
            return 0
        lax.fori_loop(0, bw // nl, mul_body, 0)

        for l in range(nlb):
            pltpu.sync_copy(pczv[l], pcz_out.at[l, pl.ds(base, bw)])
            pltpu.sync_copy(qzv[l], qz_out.at[l, pl.ds(base, bw)])
            pltpu.sync_copy(pzv[l], pz_out.at[l, pl.ds(base, bw)])

    return sc_kernel


def kernel(cells, genes, emb_table, ct_W, dec_W, dec_b, pz_cw):
    cells = cells.astype(jnp.int32)
    genes = genes.astype(jnp.int32)
    emb_t = emb_table.T
    pzcw_flat = jnp.transpose(pz_cw, (0, 2, 1)).reshape(-1)
    p_table, pc_zwT = _tc_tables(emb_t, ct_W, dec_W, dec_b)
    p_flat = p_table.reshape(-1)
    pc_flat = pc_zwT.T.reshape(-1)
    sc = _make_sc_gather()
    pczT, qzT, pzT = sc(cells, genes, pzcw_flat, p_flat, pc_flat)
    return (pczT.T, qzT.T, pzT.T)


# speedup vs baseline: 18.6783x; 1.1911x over previous
import functools

import jax
import jax.numpy as jnp
from jax import lax
from jax.experimental import pallas as pl
from jax.experimental.pallas import tpu as pltpu
from jax.experimental.pallas import tpu_sc as plsc

_N_CELLS = 50000
_N_GENES = 128
_N_LABELS = 8
_EMB = 64
_B = 16384

_BKC = 4096


def _tc_tables_body(cells_ref, genes_ref, embt_ref, ctw_ref, decw_ref,
                    decb_ref, p_ref, qidx_ref, pcr_ref):
    i = pl.program_id(0)
    ctw = ctw_ref[...]
    logits = lax.dot_general(
        embt_ref[...], ctw, (((0,), (1,)), ((), ())),
        preferred_element_type=jnp.float32)
    m = jnp.max(logits, axis=1, keepdims=True)
    e = jnp.exp(logits - m)
    p = e / jnp.sum(e, axis=1, keepdims=True)
    p_ref[...] = jnp.concatenate([p, p], axis=1)

    @pl.when(i < _B // _BKC)
    def _():
        c = cells_ref[0, pl.ds(i * _BKC, _BKC)]
        g = genes_ref[0, pl.ds(i * _BKC, _BKC)]
        l8 = lax.broadcasted_iota(jnp.int32, (_BKC, _N_LABELS), 1)
        qidx_ref[...] = c[:, None] * 1024 + l8 * 128 + g[:, None]

        dec = lax.dot_general(
            ctw, decw_ref[...], (((1,), (1,)), ((), ())),
            preferred_element_type=jnp.float32)
        dec = dec + decb_ref[...]
        m0 = jnp.max(dec, axis=1, keepdims=True)
        e0 = jnp.exp(dec - m0)
        pcT = e0 / jnp.sum(e0, axis=1, keepdims=True)

        oh = (g[:, None]
              == lax.broadcasted_iota(jnp.int32, (_BKC, _N_GENES), 1))
        pcr = lax.dot_general(
            oh.astype(jnp.float32), pcT, (((1,), (1,)), ((), ())),
            preferred_element_type=jnp.float32)
        pcr_ref[...] = jnp.concatenate([pcr, pcr], axis=1)


def _tc_tables(cells, genes, emb_t, ct_W, dec_W, dec_b):
    grid = (_N_CELLS + _BKC - 1) // _BKC
    nb = _B // _BKC
    bclamp = lambda i: (jnp.minimum(i, nb - 1), 0)
    return pl.pallas_call(
        _tc_tables_body,
        grid=(grid,),
        in_specs=[
            pl.BlockSpec((1, _B), lambda i: (0, 0)),
            pl.BlockSpec((1, _B), lambda i: (0, 0)),
            pl.BlockSpec((_EMB, _BKC), lambda i: (0, i)),
            pl.BlockSpec((_N_LABELS, _EMB), lambda i: (0, 0)),
            pl.BlockSpec((_N_GENES, _EMB), lambda i: (0, 0)),
            pl.BlockSpec((1, _N_GENES), lambda i: (0, 0)),
        ],
        out_specs=[
            pl.BlockSpec((_BKC, 2 * _N_LABELS), lambda i: (i, 0)),
            pl.BlockSpec((_BKC, _N_LABELS), bclamp),
            pl.BlockSpec((_BKC, 2 * _N_LABELS), bclamp),
        ],
        out_shape=[
            jax.ShapeDtypeStruct((_N_CELLS, 2 * _N_LABELS), jnp.float32),
            jax.ShapeDtypeStruct((_B, _N_LABELS), jnp.int32),
            jax.ShapeDtypeStruct((_B, 2 * _N_LABELS), jnp.float32),
        ],
    )(cells.reshape(1, _B), genes.reshape(1, _B), emb_t, ct_W, dec_W,
      dec_b.reshape(1, _N_GENES))


def _make_sc_gather():
    info = plsc.get_sparse_core_info()
    nc, ns, nl = info.num_cores, info.num_subcores, info.num_lanes
    nw = nc * ns
    bw = _B // nw
    nlb = _N_LABELS
    rows = bw * nlb // nl

    mesh = plsc.VectorSubcoreMesh(core_axis_name="c", subcore_axis_name="s")

    scratch = [
        pltpu.VMEM((bw,), jnp.int32),
        pltpu.VMEM((bw * nlb,), jnp.int32),
        pltpu.VMEM((bw, nl), jnp.float32),
        pltpu.VMEM((bw, nl), jnp.float32),
        pltpu.VMEM((bw * nlb,), jnp.float32),
        pltpu.VMEM((bw, nl), jnp.float32),
        pltpu.SemaphoreType.DMA,
    ]

    @functools.partial(
        pl.kernel,
        mesh=mesh,
        compiler_params=pltpu.CompilerParams(use_tc_tiling_on_sc=False),
        out_type=[
            jax.ShapeDtypeStruct((_B, nl), jnp.float32),
            jax.ShapeDtypeStruct((_B * nlb,), jnp.float32),
            jax.ShapeDtypeStruct((_B, nl), jnp.float32),
        ],
        scratch_types=scratch,
    )
    def sc_kernel(cells_hbm, qidx_hbm, pcr_hbm, pzcw_hbm, p_hbm,
                  pcz_out, qz_out, pz_out,
                  cells_v, qidx_v, pz_v, pcr_v, qz_v, pcz_v, sem):
        wid = lax.axis_index("s") * nc + lax.axis_index("c")
        base = wid * bw

        pltpu.sync_copy(cells_hbm.at[pl.ds(base, bw)], cells_v)
        cp_pz = pltpu.async_copy(p_hbm.at[cells_v], pz_v, sem)

        pltpu.sync_copy(qidx_hbm.at[pl.ds(base * nlb, bw * nlb)], qidx_v)
        cp_qz = pltpu.async_copy(pzcw_hbm.at[qidx_v], qz_v, sem)
        cp_pcr = pltpu.async_copy(pcr_hbm.at[pl.ds(base, bw)], pcr_v, sem)

        cp_pz.wait()
        cp_pcr.wait()

        def mul_body(k, _):
            pcz_v[k, :] = pz_v[k, :] * pcr_v[k, :]
            return 0
        lax.fori_loop(0, bw, mul_body, 0)

        pltpu.sync_copy(pz_v, pz_out.at[pl.ds(base, bw)])
        pltpu.sync_copy(pcz_v, pcz_out.at[pl.ds(base, bw)])
        cp_qz.wait()
        pltpu.sync_copy(qz_v, qz_out.at[pl.ds(base * nlb, bw * nlb)])

    return sc_kernel


def kernel(cells, genes, emb_table, ct_W, dec_W, dec_b, pz_cw):
    cells = cells.astype(jnp.int32)
    genes = genes.astype(jnp.int32)
    emb_t = emb_table.T
    pzcw_flat = jnp.transpose(pz_cw, (0, 2, 1)).reshape(-1)
    p_table, qidx, pc_rows = _tc_tables(cells, genes, emb_t, ct_W, dec_W,
                                        dec_b)
    sc = _make_sc_gather()
    pcz16, qz_f, pz16 = sc(cells, qidx.reshape(-1), pc_rows, pzcw_flat,
                           p_table)
    shp = (_B, _N_LABELS)
    return (pcz16[:, :_N_LABELS], qz_f.reshape(shp), pz16[:, :_N_LABELS])
